# baseline (speedup 1.0000x reference)
import jax
import jax.numpy as jnp
from jax.experimental import pallas as pl

B, N, C, K = 16, 20000, 80, 300


def _sigmoid_body(logits_ref, out_ref):
    out_ref[...] = jax.nn.sigmoid(logits_ref[...])


def kernel(pred_logits, pred_boxes, orig_target_sizes):
    flat = pred_logits.reshape(B, 200, 8000)
    scores = pl.pallas_call(
        _sigmoid_body,
        out_shape=jax.ShapeDtypeStruct((B, 200, 8000), jnp.float32),
        grid=(B,),
        in_specs=[pl.BlockSpec((1, 200, 8000), lambda b: (b, 0, 0))],
        out_specs=pl.BlockSpec((1, 200, 8000), lambda b: (b, 0, 0)),
    )(flat).reshape(B, N * C)

    top_scores, index = jax.lax.top_k(scores, K)
    labels = index % C
    qindex = index // C

    cx = pred_boxes[..., 0]
    cy = pred_boxes[..., 1]
    w = pred_boxes[..., 2]
    h = pred_boxes[..., 3]
    bbox = jnp.stack(
        [cx - 0.5 * w, cy - 0.5 * h, cx + 0.5 * w, cy + 0.5 * h], axis=-1
    )
    scale = jnp.tile(orig_target_sizes, (1, 2))[:, None, :]
    bbox = bbox * scale
    boxes = jnp.take_along_axis(bbox, qindex[..., None], axis=1)
    return (labels, boxes, top_scores)
